# baseline (speedup 1.0000x reference)
import functools
import numpy as np
import jax
import jax.numpy as jnp
from jax import lax
from jax.experimental import pallas as pl
from jax.experimental.pallas import tpu as pltpu
from jax.experimental.pallas import tpu_sc as plsc

_z = np.linspace(-12.0, 12.0, 200001)
_pdf = np.exp(-_z ** 2 / 2.0) / np.sqrt(2.0 * np.pi)
_silu_z = _z / (1.0 + np.exp(-_z))
_SILU_C = float(1.0 / np.sqrt(np.trapz(_silu_z ** 2 * _pdf, _z)))

_N_NODES = 10000
_N_EDGES = 320000
_D = 128
_INV_S8 = float(1.0 / np.sqrt(8.0))
_INV_S128 = float(1.0 / np.sqrt(128.0))
_INV_S512 = float(1.0 / np.sqrt(512.0))

_F = np.zeros(128, dtype=np.int32)
for _s in range(64):
    _F[_s] = 32 * (_s // 16) + _s % 16
    _F[64 + _s] = 32 * (_s // 16) + 16 + _s % 16

_S9 = np.zeros((144, 128), dtype=np.float32)
for _m in range(16):
    _S9[9 * _m + 8, 8 * _m:8 * _m + 8] = 1.0

_BQ = 2000
_BE2 = 2000
_BN = 2000


def _pack_i32(x_f32):
    u = lax.bitcast_convert_type(x_f32.astype(jnp.bfloat16), jnp.uint16)
    lo = u[:, :64].astype(jnp.uint32)
    hi = u[:, 64:].astype(jnp.uint32)
    return lax.bitcast_convert_type((hi << 16) | lo, jnp.int32)


def _edge_w_body(x_ref, w1_ref, s_ref, m_ref, out_ref):
    xb = x_ref[...].astype(jnp.bfloat16)
    hp = jnp.dot(xb, w1_ref[...],
                 preferred_element_type=jnp.float32) * _INV_S8
    a = jnp.dot(xb, s_ref[...], preferred_element_type=jnp.float32)
    h = (hp * (_SILU_C / (1.0 + jnp.exp(-hp))) * a).astype(jnp.bfloat16)
    for p in range(8):
        x = jnp.dot(h, m_ref[p], preferred_element_type=jnp.float32) * _INV_S8
        out_ref[:, p, :] = jnp.concatenate(
            [_pack_i32(x[:, :_D]), _pack_i32(x[:, _D:])], axis=1)


def _edge_weights(emb9, W1bd_bf, S9_bf, M8_bf):
    grid = (_N_EDGES // 16) // _BQ
    return pl.pallas_call(
        _edge_w_body,
        grid=(grid,),
        in_specs=[
            pl.BlockSpec((_BQ, 144), lambda i: (i, 0)),
            pl.BlockSpec((144, _D), lambda i: (0, 0)),
            pl.BlockSpec((144, _D), lambda i: (0, 0)),
            pl.BlockSpec((8, _D, 2 * _D), lambda i: (0, 0, 0)),
        ],
        out_specs=pl.BlockSpec((_BQ, 8, _D), lambda i: (i, 0, 0)),
        out_shape=jax.ShapeDtypeStruct((_N_EDGES // 16, 8, _D), jnp.int32),
    )(emb9, W1bd_bf, S9_bf, M8_bf)


def _nf_body(x_ref, w_ref, o_ref):
    o_ref[...] = jnp.dot(x_ref[...].astype(jnp.bfloat16), w_ref[...],
                         preferred_element_type=jnp.float32) * _INV_S128


def _nf(node_features, W1p_bf):
    grid = _N_NODES // _BN
    return pl.pallas_call(
        _nf_body,
        grid=(grid,),
        in_specs=[
            pl.BlockSpec((_BN, _D), lambda i: (i, 0)),
            pl.BlockSpec((_D, _D), lambda i: (0, 0)),
        ],
        out_specs=pl.BlockSpec((_BN, _D), lambda i: (i, 0)),
        out_shape=jax.ShapeDtypeStruct((_N_NODES, _D), jnp.float32),
    )(node_features, W1p_bf)


def _finish_body(agg_ref, x_ref, a_ref, wlin2_ref, wsc_ref, o_ref):
    agg = agg_ref[0] + agg_ref[1]
    acc = jnp.dot(agg, wlin2_ref[...]) * _INV_S128
    x = x_ref[...]
    a = a_ref[...]
    for v in range(4):
        acc = acc + jnp.dot(x, wsc_ref[v]) * (a[:, v:v + 1] * _INV_S512)
    o_ref[...] = acc


def _finish(agg2, node_features, node_attrs, W_lin2, W_sc_t):
    grid = _N_NODES // _BN
    return pl.pallas_call(
        _finish_body,
        grid=(grid,),
        in_specs=[
            pl.BlockSpec((2, _BN, _D), lambda i: (0, i, 0)),
            pl.BlockSpec((_BN, _D), lambda i: (i, 0)),
            pl.BlockSpec((_BN, 4), lambda i: (i, 0)),
            pl.BlockSpec((_D, _D), lambda i: (0, 0)),
            pl.BlockSpec((4, _D, _D), lambda i: (0, 0, 0)),
        ],
        out_specs=pl.BlockSpec((_BN, _D), lambda i: (i, 0)),
        out_shape=jax.ShapeDtypeStruct((_N_NODES, _D), jnp.float32),
    )(agg2, node_features, node_attrs, W_lin2, W_sc_t)


_NC = 2
_NS = 16
_NW = _NC * _NS
_EPW = _N_EDGES // _NW
_CE = 80
_NCHUNK = _EPW // _CE
_NPAD = 10240
_RPS = _NPAD // _NS
_HIMASK = np.int32(-65536)


def _unpack2(v):
    lo = lax.bitcast_convert_type(v << 16, jnp.float32)
    hi = lax.bitcast_convert_type(v & _HIMASK, jnp.float32)
    return lo, hi


def _mul_half(rows_f, w_i, prod, q0, q1):
    @plsc.parallel_loop(q0, q1, unroll=2)
    def _grp(qi):
        rb = 16 * qi
        for p in range(8):
            for side in range(2):
                r = rb + 2 * p + side
                for k in range(4):
                    vw = w_i[qi, p, pl.ds(64 * side + 16 * k, 16)]
                    wlo, whi = _unpack2(vw)
                    prod[r, pl.ds(32 * k, 16)] = (
                        rows_f[r, pl.ds(32 * k, 16)] * wlo)
                    prod[r, pl.ds(32 * k + 16, 16)] = (
                        rows_f[r, pl.ds(32 * k + 16, 16)] * whi)


def _sc_body(nbr_hbm, ctr_hbm, nf_hbm, w_hbm, out_hbm,
             nbr0, nbr1, ctrA0, ctrA1, ctrB0, ctrB1,
             rows0, rows1, wv0, wv1, prod,
             sem_in0, sem_in1, sem_icA0, sem_icA1, sem_icB0, sem_icB1,
             sem_g0, sem_g1, sem_w0, sem_w1, sem_sA, sem_sB,
             agg_sh):
    cid = lax.axis_index("c")
    sid = lax.axis_index("s")
    wid = cid * _NS + sid
    base_w = wid * _EPW
    base_q = wid * (_EPW // 16)

    nbr = (nbr0, nbr1)
    ctrA = (ctrA0, ctrA1)
    ctrB = (ctrB0, ctrB1)
    rows = (rows0, rows1)
    wv = (wv0, wv1)
    sem_in = (sem_in0, sem_in1)
    sem_icA = (sem_icA0, sem_icA1)
    sem_icB = (sem_icB0, sem_icB1)
    sem_g = (sem_g0, sem_g1)
    sem_w = (sem_w0, sem_w1)

    @pl.loop(0, _CE)
    def _zrow(r):
        for k in range(8):
            prod[r, pl.ds(16 * k, 16)] = jnp.zeros((16,), jnp.float32)

    @pl.loop(0, _RPS // _CE)
    def _zero(j):
        pltpu.sync_copy(prod, agg_sh.at[pl.ds(sid * _RPS + j * _CE, _CE)])

    plsc.subcore_barrier()

    pltpu.sync_copy(nbr_hbm.at[pl.ds(base_w, _CE)], nbr0)
    pltpu.async_copy(nbr_hbm.at[pl.ds(base_w + _CE, _CE)], nbr1, sem_in1)
    pltpu.async_copy(nf_hbm.at[nbr0], rows0, sem_g0)
    pltpu.async_copy(w_hbm.at[pl.ds(base_q, _CE // 16)], wv0, sem_w0)

    def _chunk_step(ch, b):
        o = 1 - b
        @pl.when(ch + 1 < _NCHUNK)
        def _issue_next():
            pltpu.make_async_copy(nbr_hbm.at[pl.ds(0, _CE)], nbr[o],
                                  sem_in[o]).wait()
            pltpu.async_copy(nf_hbm.at[nbr[o]], rows[o], sem_g[o])
            pltpu.async_copy(
                w_hbm.at[pl.ds(base_q + (ch + 1) * (_CE // 16), _CE // 16)],
                wv[o], sem_w[o])

        pltpu.make_async_copy(nf_hbm.at[nbr[b]], rows[b], sem_g[b]).wait()
        pltpu.make_async_copy(w_hbm.at[pl.ds(0, _CE // 16)], wv[b],
                              sem_w[b]).wait()

        @pl.when(ch + 2 < _NCHUNK)
        def _issue_nbr():
            pltpu.async_copy(
                nbr_hbm.at[pl.ds(base_w + (ch + 2) * _CE, _CE)],
                nbr[b], sem_in[b])

        @pl.when(ch >= 1)
        def _drain_scat():
            pltpu.make_async_copy(prod, agg_sh.at[ctrA[b]], sem_sA).wait()

        pltpu.async_copy(ctr_hbm.at[pl.ds(base_w + ch * _CE, _CE)],
                         ctrA[b], sem_icA[b])

        _mul_half(rows[b], wv[b], prod, 0, 5)

        pltpu.make_async_copy(ctr_hbm.at[pl.ds(0, _CE)], ctrA[b],
                              sem_icA[b]).wait()
        pltpu.async_copy(prod, agg_sh.at[ctrA[b]], sem_sA, add=True)

    @pl.loop(0, _NCHUNK // 2)
    def _pair(p):
        _chunk_step(2 * p, 0)
        _chunk_step(2 * p + 1, 1)

    _chunk_step(_NCHUNK - 1, 0)

    pltpu.make_async_copy(prod, agg_sh.at[ctrA[0]], sem_sA).wait()

    plsc.subcore_barrier()

    pltpu.sync_copy(agg_sh.at[pl.ds(sid * _RPS, _RPS)],
                    out_hbm.at[pl.ds(cid * _NPAD + sid * _RPS, _RPS)])


@functools.lru_cache(maxsize=1)
def _make_sc_call():
    return functools.partial(
        pl.kernel,
        out_type=jax.ShapeDtypeStruct((_NC * _NPAD, _D), jnp.float32),
        mesh=plsc.VectorSubcoreMesh(core_axis_name="c", subcore_axis_name="s"),
        scratch_types=[
            pltpu.VMEM((_CE,), jnp.int32),
            pltpu.VMEM((_CE,), jnp.int32),
            pltpu.VMEM((_CE,), jnp.int32),
            pltpu.VMEM((_CE,), jnp.int32),
            pltpu.VMEM((32,), jnp.int32),
            pltpu.VMEM((32,), jnp.int32),
            pltpu.VMEM((_CE, _D), jnp.float32),
            pltpu.VMEM((_CE, _D), jnp.float32),
            pltpu.VMEM((_CE // 16, 8, _D), jnp.int32),
            pltpu.VMEM((_CE // 16, 8, _D), jnp.int32),
            pltpu.VMEM((_CE, _D), jnp.float32),
            pltpu.SemaphoreType.DMA,
            pltpu.SemaphoreType.DMA,
            pltpu.SemaphoreType.DMA,
            pltpu.SemaphoreType.DMA,
            pltpu.SemaphoreType.DMA,
            pltpu.SemaphoreType.DMA,
            pltpu.SemaphoreType.DMA,
            pltpu.SemaphoreType.DMA,
            pltpu.SemaphoreType.DMA,
            pltpu.SemaphoreType.DMA,
            pltpu.SemaphoreType.DMA,
            pltpu.SemaphoreType.DMA,
            pltpu.VMEM_SHARED((_NPAD, _D), jnp.float32),
        ],
    )(_sc_body)


def kernel(edge_embedding, node_attrs, node_features, edge_index, edge_attrs,
           W_lin1, W_fc1, W_fc2, W_lin2, W_sc):
    ei = edge_index.astype(jnp.int32)
    center = ei[0]
    neighbor = ei[1]

    perm = jnp.asarray(_F)
    W1bd9 = jnp.zeros((144, _D), jnp.float32)
    for m in range(16):
        W1bd9 = W1bd9.at[9 * m:9 * m + 8, 8 * m:8 * m + 8].set(W_fc1)
    W1bd_bf = W1bd9.astype(jnp.bfloat16)
    W2s = W_fc2[:, perm]
    M8 = jnp.zeros((8, _D, 2 * _D), jnp.float32)
    for p in range(8):
        M8 = M8.at[p, 16 * p:16 * p + 8, :_D].set(W2s)
        M8 = M8.at[p, 16 * p + 8:16 * p + 16, _D:].set(W2s)
    M8_bf = M8.astype(jnp.bfloat16)
    W1p_bf = W_lin1.astype(jnp.bfloat16)

    emb9 = jnp.concatenate([edge_embedding, edge_attrs],
                           axis=1).reshape(_N_EDGES // 16, 144)
    w_edge = _edge_weights(emb9, W1bd_bf,
                           jnp.asarray(_S9, dtype=jnp.bfloat16), M8_bf)
    nf = _nf(node_features, W1p_bf)

    agg2 = _make_sc_call()(neighbor, center, nf,
                           w_edge).reshape(_NC, _NPAD, _D)

    return _finish(agg2, node_features, node_attrs, W_lin2,
                   jnp.transpose(W_sc, (1, 0, 2)))


# speedup vs baseline: 1.0697x; 1.0697x over previous
import functools
import numpy as np
import jax
import jax.numpy as jnp
from jax import lax
from jax.experimental import pallas as pl
from jax.experimental.pallas import tpu as pltpu
from jax.experimental.pallas import tpu_sc as plsc

_z = np.linspace(-12.0, 12.0, 200001)
_pdf = np.exp(-_z ** 2 / 2.0) / np.sqrt(2.0 * np.pi)
_silu_z = _z / (1.0 + np.exp(-_z))
_SILU_C = float(1.0 / np.sqrt(np.trapz(_silu_z ** 2 * _pdf, _z)))

_N_NODES = 10000
_N_EDGES = 320000
_D = 128
_INV_S8 = float(1.0 / np.sqrt(8.0))
_INV_S128 = float(1.0 / np.sqrt(128.0))
_INV_S512 = float(1.0 / np.sqrt(512.0))

_F = np.zeros(128, dtype=np.int32)
for _s in range(64):
    _F[_s] = 32 * (_s // 16) + _s % 16
    _F[64 + _s] = 32 * (_s // 16) + 16 + _s % 16

_S9 = np.zeros((144, 128), dtype=np.float32)
for _m in range(16):
    _S9[9 * _m + 8, 8 * _m:8 * _m + 8] = 1.0

_BQ = 2000
_BE2 = 2000
_BN = 2000


def _pack_i32(x_f32):
    u = lax.bitcast_convert_type(x_f32.astype(jnp.bfloat16), jnp.uint16)
    lo = u[:, :64].astype(jnp.uint32)
    hi = u[:, 64:].astype(jnp.uint32)
    return lax.bitcast_convert_type((hi << 16) | lo, jnp.int32)


def _edge_w_body(x_ref, w1_ref, s_ref, m_ref, out_ref):
    xb = x_ref[...].astype(jnp.bfloat16)
    hp = jnp.dot(xb, w1_ref[...],
                 preferred_element_type=jnp.float32) * _INV_S8
    a = jnp.dot(xb, s_ref[...], preferred_element_type=jnp.float32)
    h = (hp * (_SILU_C / (1.0 + jnp.exp(-hp))) * a).astype(jnp.bfloat16)
    for p in range(8):
        x = jnp.dot(h, m_ref[p], preferred_element_type=jnp.float32) * _INV_S8
        out_ref[:, p, :] = jnp.concatenate(
            [_pack_i32(x[:, :_D]), _pack_i32(x[:, _D:])], axis=1)


def _edge_weights(emb9, W1bd_bf, S9_bf, M8_bf):
    grid = (_N_EDGES // 16) // _BQ
    return pl.pallas_call(
        _edge_w_body,
        grid=(grid,),
        in_specs=[
            pl.BlockSpec((_BQ, 144), lambda i: (i, 0)),
            pl.BlockSpec((144, _D), lambda i: (0, 0)),
            pl.BlockSpec((144, _D), lambda i: (0, 0)),
            pl.BlockSpec((8, _D, 2 * _D), lambda i: (0, 0, 0)),
        ],
        out_specs=pl.BlockSpec((_BQ, 8, _D), lambda i: (i, 0, 0)),
        out_shape=jax.ShapeDtypeStruct((_N_EDGES // 16, 8, _D), jnp.int32),
    )(emb9, W1bd_bf, S9_bf, M8_bf)


def _nf_body(x_ref, w_ref, o_ref):
    o_ref[...] = jnp.dot(x_ref[...].astype(jnp.bfloat16), w_ref[...],
                         preferred_element_type=jnp.float32) * _INV_S128


def _nf(node_features, W1p_bf):
    grid = _N_NODES // _BN
    return pl.pallas_call(
        _nf_body,
        grid=(grid,),
        in_specs=[
            pl.BlockSpec((_BN, _D), lambda i: (i, 0)),
            pl.BlockSpec((_D, _D), lambda i: (0, 0)),
        ],
        out_specs=pl.BlockSpec((_BN, _D), lambda i: (i, 0)),
        out_shape=jax.ShapeDtypeStruct((_N_NODES, _D), jnp.float32),
    )(node_features, W1p_bf)


def _finish_body(agg_ref, x_ref, a_ref, wlin2_ref, wsc_ref, o_ref):
    agg = agg_ref[0] + agg_ref[1]
    acc = jnp.dot(agg, wlin2_ref[...]) * _INV_S128
    x = x_ref[...]
    a = a_ref[...]
    for v in range(4):
        acc = acc + jnp.dot(x, wsc_ref[v]) * (a[:, v:v + 1] * _INV_S512)
    o_ref[...] = acc


def _finish(agg2, node_features, node_attrs, W_lin2, W_sc_t):
    grid = _N_NODES // _BN
    return pl.pallas_call(
        _finish_body,
        grid=(grid,),
        in_specs=[
            pl.BlockSpec((2, _BN, _D), lambda i: (0, i, 0)),
            pl.BlockSpec((_BN, _D), lambda i: (i, 0)),
            pl.BlockSpec((_BN, 4), lambda i: (i, 0)),
            pl.BlockSpec((_D, _D), lambda i: (0, 0)),
            pl.BlockSpec((4, _D, _D), lambda i: (0, 0, 0)),
        ],
        out_specs=pl.BlockSpec((_BN, _D), lambda i: (i, 0)),
        out_shape=jax.ShapeDtypeStruct((_N_NODES, _D), jnp.float32),
    )(agg2, node_features, node_attrs, W_lin2, W_sc_t)


_NC = 2
_NS = 16
_NW = _NC * _NS
_EPW = _N_EDGES // _NW
_CE = 80
_NCHUNK = _EPW // _CE
_NPAD = 10240
_RPS = _NPAD // _NS
_HIMASK = np.int32(-65536)


def _unpack2(v):
    lo = lax.bitcast_convert_type(v << 16, jnp.float32)
    hi = lax.bitcast_convert_type(v & _HIMASK, jnp.float32)
    return lo, hi


def _mul_half(rows_f, w_i, prod, q0, q1):
    @plsc.parallel_loop(q0, q1, unroll=1)
    def _grp(qi):
        rb = 16 * qi
        for p in range(8):
            for side in range(2):
                r = rb + 2 * p + side
                for k in range(4):
                    vw = w_i[qi, p, pl.ds(64 * side + 16 * k, 16)]
                    wlo, whi = _unpack2(vw)
                    prod[r, pl.ds(32 * k, 16)] = (
                        rows_f[r, pl.ds(32 * k, 16)] * wlo)
                    prod[r, pl.ds(32 * k + 16, 16)] = (
                        rows_f[r, pl.ds(32 * k + 16, 16)] * whi)


def _sc_body(nbr_hbm, ctr_hbm, nf_hbm, w_hbm, out_hbm,
             nbr0, nbr1, ctrA0, ctrA1, ctrB0, ctrB1,
             rows0, rows1, wv0, wv1, prod,
             sem_in0, sem_in1, sem_icA0, sem_icA1, sem_icB0, sem_icB1,
             sem_g0, sem_g1, sem_w0, sem_w1, sem_sA, sem_sB,
             agg_sh):
    cid = lax.axis_index("c")
    sid = lax.axis_index("s")
    wid = cid * _NS + sid
    base_w = wid * _EPW
    base_q = wid * (_EPW // 16)

    nbr = (nbr0, nbr1)
    ctrA = (ctrA0, ctrA1)
    ctrB = (ctrB0, ctrB1)
    rows = (rows0, rows1)
    wv = (wv0, wv1)
    sem_in = (sem_in0, sem_in1)
    sem_icA = (sem_icA0, sem_icA1)
    sem_icB = (sem_icB0, sem_icB1)
    sem_g = (sem_g0, sem_g1)
    sem_w = (sem_w0, sem_w1)

    @pl.loop(0, _CE)
    def _zrow(r):
        for k in range(8):
            prod[r, pl.ds(16 * k, 16)] = jnp.zeros((16,), jnp.float32)

    @pl.loop(0, _RPS // _CE)
    def _zero(j):
        pltpu.sync_copy(prod, agg_sh.at[pl.ds(sid * _RPS + j * _CE, _CE)])

    plsc.subcore_barrier()

    pltpu.sync_copy(nbr_hbm.at[pl.ds(base_w, _CE)], nbr0)
    pltpu.async_copy(nbr_hbm.at[pl.ds(base_w + _CE, _CE)], nbr1, sem_in1)
    pltpu.async_copy(nf_hbm.at[nbr0], rows0, sem_g0)
    pltpu.async_copy(w_hbm.at[pl.ds(base_q, _CE // 16)], wv0, sem_w0)

    def _chunk_step(ch, b):
        o = 1 - b
        @pl.when(ch + 1 < _NCHUNK)
        def _issue_next():
            pltpu.make_async_copy(nbr_hbm.at[pl.ds(0, _CE)], nbr[o],
                                  sem_in[o]).wait()
            pltpu.async_copy(nf_hbm.at[nbr[o]], rows[o], sem_g[o])
            pltpu.async_copy(
                w_hbm.at[pl.ds(base_q + (ch + 1) * (_CE // 16), _CE // 16)],
                wv[o], sem_w[o])

        pltpu.make_async_copy(nf_hbm.at[nbr[b]], rows[b], sem_g[b]).wait()
        pltpu.make_async_copy(w_hbm.at[pl.ds(0, _CE // 16)], wv[b],
                              sem_w[b]).wait()

        @pl.when(ch + 2 < _NCHUNK)
        def _issue_nbr():
            pltpu.async_copy(
                nbr_hbm.at[pl.ds(base_w + (ch + 2) * _CE, _CE)],
                nbr[b], sem_in[b])

        @pl.when(ch >= 1)
        def _drain_scat():
            pltpu.make_async_copy(prod, agg_sh.at[ctrA[b]], sem_sA).wait()

        pltpu.async_copy(ctr_hbm.at[pl.ds(base_w + ch * _CE, _CE)],
                         ctrA[b], sem_icA[b])

        _mul_half(rows[b], wv[b], prod, 0, 5)

        pltpu.make_async_copy(ctr_hbm.at[pl.ds(0, _CE)], ctrA[b],
                              sem_icA[b]).wait()
        pltpu.async_copy(prod, agg_sh.at[ctrA[b]], sem_sA, add=True)

    @pl.loop(0, _NCHUNK // 2)
    def _pair(p):
        _chunk_step(2 * p, 0)
        _chunk_step(2 * p + 1, 1)

    _chunk_step(_NCHUNK - 1, 0)

    pltpu.make_async_copy(prod, agg_sh.at[ctrA[0]], sem_sA).wait()

    plsc.subcore_barrier()

    pltpu.sync_copy(agg_sh.at[pl.ds(sid * _RPS, _RPS)],
                    out_hbm.at[pl.ds(cid * _NPAD + sid * _RPS, _RPS)])


@functools.lru_cache(maxsize=1)
def _make_sc_call():
    return functools.partial(
        pl.kernel,
        out_type=jax.ShapeDtypeStruct((_NC * _NPAD, _D), jnp.float32),
        mesh=plsc.VectorSubcoreMesh(core_axis_name="c", subcore_axis_name="s"),
        scratch_types=[
            pltpu.VMEM((_CE,), jnp.int32),
            pltpu.VMEM((_CE,), jnp.int32),
            pltpu.VMEM((_CE,), jnp.int32),
            pltpu.VMEM((_CE,), jnp.int32),
            pltpu.VMEM((32,), jnp.int32),
            pltpu.VMEM((32,), jnp.int32),
            pltpu.VMEM((_CE, _D), jnp.float32),
            pltpu.VMEM((_CE, _D), jnp.float32),
            pltpu.VMEM((_CE // 16, 8, _D), jnp.int32),
            pltpu.VMEM((_CE // 16, 8, _D), jnp.int32),
            pltpu.VMEM((_CE, _D), jnp.float32),
            pltpu.SemaphoreType.DMA,
            pltpu.SemaphoreType.DMA,
            pltpu.SemaphoreType.DMA,
            pltpu.SemaphoreType.DMA,
            pltpu.SemaphoreType.DMA,
            pltpu.SemaphoreType.DMA,
            pltpu.SemaphoreType.DMA,
            pltpu.SemaphoreType.DMA,
            pltpu.SemaphoreType.DMA,
            pltpu.SemaphoreType.DMA,
            pltpu.SemaphoreType.DMA,
            pltpu.SemaphoreType.DMA,
            pltpu.VMEM_SHARED((_NPAD, _D), jnp.float32),
        ],
    )(_sc_body)


def kernel(edge_embedding, node_attrs, node_features, edge_index, edge_attrs,
           W_lin1, W_fc1, W_fc2, W_lin2, W_sc):
    ei = edge_index.astype(jnp.int32)
    center = ei[0]
    neighbor = ei[1]

    perm = jnp.asarray(_F)
    W1bd9 = jnp.zeros((144, _D), jnp.float32)
    for m in range(16):
        W1bd9 = W1bd9.at[9 * m:9 * m + 8, 8 * m:8 * m + 8].set(W_fc1)
    W1bd_bf = W1bd9.astype(jnp.bfloat16)
    W2s = W_fc2[:, perm]
    M8 = jnp.zeros((8, _D, 2 * _D), jnp.float32)
    for p in range(8):
        M8 = M8.at[p, 16 * p:16 * p + 8, :_D].set(W2s)
        M8 = M8.at[p, 16 * p + 8:16 * p + 16, _D:].set(W2s)
    M8_bf = M8.astype(jnp.bfloat16)
    W1p_bf = W_lin1.astype(jnp.bfloat16)

    emb9 = jnp.concatenate([edge_embedding, edge_attrs],
                           axis=1).reshape(_N_EDGES // 16, 144)
    w_edge = _edge_weights(emb9, W1bd_bf,
                           jnp.asarray(_S9, dtype=jnp.bfloat16), M8_bf)
    nf = _nf(node_features, W1p_bf)

    agg2 = _make_sc_call()(neighbor, center, nf,
                           w_edge).reshape(_NC, _NPAD, _D)

    return _finish(agg2, node_features, node_attrs, W_lin2,
                   jnp.transpose(W_sc, (1, 0, 2)))
